# Initial kernel scaffold; baseline (speedup 1.0000x reference)
import jax
import jax.numpy as jnp
from jax.experimental import pallas as pl


def kernel(x, edge_index0, edge_index1, W_self0, W_neigh0, b0, W_self1, W_neigh1, b1):
    raise NotImplementedError("write your pallas kernel here")


# speedup vs baseline: 4.0938x; 4.0938x over previous
import functools

import jax
import jax.numpy as jnp
from jax import lax
from jax.experimental import pallas as pl
from jax.experimental.pallas import tpu as pltpu
from jax.experimental.pallas import tpu_sc as plsc

N_NODES = 10000
NUM_CORES = 2
NUM_SUBCORES = 16
CHUNK = 104
CHUNKS_PER_W = 98
ACC_ROWS = 10016
ZERO_ROWS = 640
LAST_ROWS = ACC_ROWS - 15 * ZERO_ROWS
E_PER_W = CHUNKS_PER_W * CHUNK
E_PAD = NUM_CORES * NUM_SUBCORES * E_PER_W


def _segsum_sc(feat, src, dst, zeros, d):
    mesh = plsc.VectorSubcoreMesh(
        core_axis_name="c", subcore_axis_name="s",
        num_cores=NUM_CORES, num_subcores=NUM_SUBCORES,
    )

    @functools.partial(
        pl.kernel,
        out_type=jax.ShapeDtypeStruct((NUM_CORES, ACC_ROWS, d), jnp.float32),
        mesh=mesh,
        scratch_types=[
            pltpu.VMEM_SHARED((ACC_ROWS, d), jnp.float32),
            pltpu.VMEM((E_PER_W,), jnp.int32),
            pltpu.VMEM((CHUNKS_PER_W, CHUNK), jnp.int32),
            pltpu.VMEM((CHUNK, d), jnp.float32),
            pltpu.VMEM((CHUNK, d), jnp.float32),
            pltpu.SemaphoreType.DMA,
            pltpu.SemaphoreType.DMA,
        ],
    )
    def seg_kernel(feat_hbm, src_hbm, dst_hbm, zeros_hbm, out_hbm,
                   acc, src_v, dst_v, rows0, rows1, sem0, sem1):
        cid = lax.axis_index("c")
        sid = lax.axis_index("s")

        pltpu.sync_copy(src_hbm.at[cid].at[sid], src_v)
        pltpu.sync_copy(dst_hbm.at[cid].at[sid], dst_v)
        @pl.when(sid < NUM_SUBCORES - 1)
        def _():
            pltpu.sync_copy(zeros_hbm,
                            acc.at[pl.ds(sid * ZERO_ROWS, ZERO_ROWS)])

        @pl.when(sid == NUM_SUBCORES - 1)
        def _():
            pltpu.sync_copy(zeros_hbm.at[pl.ds(0, LAST_ROWS)],
                            acc.at[pl.ds(15 * ZERO_ROWS, LAST_ROWS)])

        plsc.subcore_barrier()

        def src_idx(jj):
            return src_v.at[pl.ds(jj * CHUNK, CHUNK)]

        pltpu.async_copy(feat_hbm.at[src_idx(0)], rows0, sem0)
        pltpu.async_copy(feat_hbm.at[src_idx(1)], rows1, sem1)

        def step(jj, rows, sem, issue_next):
            pltpu.make_async_copy(feat_hbm.at[pl.ds(0, CHUNK)], rows, sem).wait()
            pltpu.sync_copy(rows, acc.at[dst_v.at[jj]], add=True)
            if issue_next:
                pltpu.async_copy(feat_hbm.at[src_idx(jj + 2)], rows, sem)

        def loop_body(j, carry):
            jj = 2 * j
            step(jj, rows0, sem0, True)
            step(jj + 1, rows1, sem1, True)
            return carry

        lax.fori_loop(0, CHUNKS_PER_W // 2 - 1, loop_body, 0)
        step(CHUNKS_PER_W - 2, rows0, sem0, False)
        step(CHUNKS_PER_W - 1, rows1, sem1, False)

        plsc.subcore_barrier()

        @pl.when(sid < NUM_SUBCORES - 1)
        def _():
            pltpu.sync_copy(
                acc.at[pl.ds(sid * ZERO_ROWS, ZERO_ROWS)],
                out_hbm.at[cid].at[pl.ds(sid * ZERO_ROWS, ZERO_ROWS)],
            )

        @pl.when(sid == NUM_SUBCORES - 1)
        def _():
            pltpu.sync_copy(
                acc.at[pl.ds(15 * ZERO_ROWS, LAST_ROWS)],
                out_hbm.at[cid].at[pl.ds(15 * ZERO_ROWS, LAST_ROWS)],
            )

    return seg_kernel(feat, src, dst, zeros)


def _tc_layer0(x, acc0, W_self0, W_neigh0, b0):
    R = 1000

    def body(x_ref, a_ref, ws_ref, wn_ref, b_ref, h_ref):
        agg = a_ref[0] + a_ref[1]
        h = (
            jnp.dot(x_ref[...], ws_ref[...], preferred_element_type=jnp.float32)
            + jnp.dot(agg, wn_ref[...], preferred_element_type=jnp.float32)
            + b_ref[...]
        )
        h_ref[...] = jnp.maximum(h, 0.0)

    return pl.pallas_call(
        body,
        grid=(N_NODES // R,),
        in_specs=[
            pl.BlockSpec((R, 128), lambda i: (i, 0)),
            pl.BlockSpec((2, R, 128), lambda i: (0, i, 0)),
            pl.BlockSpec((128, 128), lambda i: (0, 0)),
            pl.BlockSpec((128, 128), lambda i: (0, 0)),
            pl.BlockSpec((1, 128), lambda i: (0, 0)),
        ],
        out_specs=pl.BlockSpec((R, 128), lambda i: (i, 0)),
        out_shape=jax.ShapeDtypeStruct((N_NODES, 128), jnp.float32),
    )(x, acc0, W_self0, W_neigh0, b0.reshape(1, 128))


def _tc_layer1(h, acc1, W_self1, W_neigh1, b1):
    R = 1000

    def body(h_ref, a_ref, ws_ref, wn_ref, b_ref, out_ref):
        agg = a_ref[0] + a_ref[1]
        out_ref[...] = (
            jnp.dot(h_ref[...], ws_ref[...], preferred_element_type=jnp.float32)
            + jnp.dot(agg, wn_ref[...], preferred_element_type=jnp.float32)
            + b_ref[...]
        )

    return pl.pallas_call(
        body,
        grid=(N_NODES // R,),
        in_specs=[
            pl.BlockSpec((R, 128), lambda i: (i, 0)),
            pl.BlockSpec((2, R, 128), lambda i: (0, i, 0)),
            pl.BlockSpec((128, 64), lambda i: (0, 0)),
            pl.BlockSpec((128, 64), lambda i: (0, 0)),
            pl.BlockSpec((1, 64), lambda i: (0, 0)),
        ],
        out_specs=pl.BlockSpec((R, 64), lambda i: (i, 0)),
        out_shape=jax.ShapeDtypeStruct((N_NODES, 64), jnp.float32),
    )(h, acc1, W_self1, W_neigh1, b1.reshape(1, 64))


def _pad_edges(edge_index):
    e = edge_index.shape[1]
    pad = E_PAD - e
    src = jnp.concatenate([edge_index[0], jnp.zeros((pad,), jnp.int32)])
    dst = jnp.concatenate(
        [edge_index[1], jnp.full((pad,), N_NODES, jnp.int32)])
    src = src.reshape(NUM_CORES, NUM_SUBCORES, E_PER_W)
    dst = dst.reshape(NUM_CORES, NUM_SUBCORES, CHUNKS_PER_W, CHUNK)
    return src, dst


def kernel(x, edge_index0, edge_index1, W_self0, W_neigh0, b0,
           W_self1, W_neigh1, b1):
    src0, dst0 = _pad_edges(edge_index0)
    src1, dst1 = _pad_edges(edge_index1)
    zeros128 = jnp.zeros((ZERO_ROWS, 128), jnp.float32)

    acc0 = _segsum_sc(x, src0, dst0, zeros128, 128)
    h = _tc_layer0(x, acc0, W_self0, W_neigh0, b0)
    acc1 = _segsum_sc(h, src1, dst1, zeros128, 128)
    return _tc_layer1(h, acc1, W_self1, W_neigh1, b1)


# speedup vs baseline: 11.4158x; 2.7886x over previous
import functools

import jax
import jax.numpy as jnp
from jax import lax
from jax.experimental import pallas as pl
from jax.experimental.pallas import tpu as pltpu
from jax.experimental.pallas import tpu_sc as plsc

N_NODES = 10000
NUM_CORES = 2
NUM_SUBCORES = 16
CHUNK = 104
CHUNKS_PER_W = 98
ACC_ROWS = 10016
ZERO_ROWS = 640
LAST_ROWS = ACC_ROWS - 15 * ZERO_ROWS
E_PER_W = CHUNKS_PER_W * CHUNK
E_PAD = NUM_CORES * NUM_SUBCORES * E_PER_W


def _segsum_sc(feat, src, dst, zeros, d):
    mesh = plsc.VectorSubcoreMesh(
        core_axis_name="c", subcore_axis_name="s",
        num_cores=NUM_CORES, num_subcores=NUM_SUBCORES,
    )

    @functools.partial(
        pl.kernel,
        out_type=jax.ShapeDtypeStruct((NUM_CORES, ACC_ROWS, d), jnp.float32),
        mesh=mesh,
        scratch_types=[
            pltpu.VMEM_SHARED((ACC_ROWS, d), jnp.float32),
            pltpu.VMEM((E_PER_W,), jnp.int32),
            pltpu.VMEM((CHUNKS_PER_W, CHUNK), jnp.int32),
            pltpu.VMEM((CHUNK, d), jnp.float32),
            pltpu.VMEM((CHUNK, d), jnp.float32),
            pltpu.SemaphoreType.DMA,
            pltpu.SemaphoreType.DMA,
        ],
    )
    def seg_kernel(feat_hbm, src_hbm, dst_hbm, zeros_hbm, out_hbm,
                   acc, src_v, dst_v, rows0, rows1, sem0, sem1):
        cid = lax.axis_index("c")
        sid = lax.axis_index("s")

        pltpu.sync_copy(src_hbm.at[cid].at[sid], src_v)
        pltpu.sync_copy(dst_hbm.at[cid].at[sid], dst_v)
        @pl.when(sid < NUM_SUBCORES - 1)
        def _():
            pltpu.sync_copy(zeros_hbm,
                            acc.at[pl.ds(sid * ZERO_ROWS, ZERO_ROWS)])

        @pl.when(sid == NUM_SUBCORES - 1)
        def _():
            pltpu.sync_copy(zeros_hbm.at[pl.ds(0, LAST_ROWS)],
                            acc.at[pl.ds(15 * ZERO_ROWS, LAST_ROWS)])

        plsc.subcore_barrier()

        def src_idx(jj):
            return src_v.at[pl.ds(jj * CHUNK, CHUNK)]

        pltpu.async_copy(feat_hbm.at[src_idx(0)], rows0, sem0)
        pltpu.async_copy(feat_hbm.at[src_idx(1)], rows1, sem1)

        def step(jj, rows, sem, issue_next):
            pltpu.make_async_copy(feat_hbm.at[pl.ds(0, CHUNK)], rows, sem).wait()
            pltpu.sync_copy(rows, acc.at[dst_v.at[jj]], add=True)
            if issue_next:
                pltpu.async_copy(feat_hbm.at[src_idx(jj + 2)], rows, sem)

        def loop_body(j, carry):
            jj = 2 * j
            step(jj, rows0, sem0, True)
            step(jj + 1, rows1, sem1, True)
            return carry

        lax.fori_loop(0, CHUNKS_PER_W // 2 - 1, loop_body, 0)
        step(CHUNKS_PER_W - 2, rows0, sem0, False)
        step(CHUNKS_PER_W - 1, rows1, sem1, False)

        plsc.subcore_barrier()

        @pl.when(sid < NUM_SUBCORES - 1)
        def _():
            pltpu.sync_copy(
                acc.at[pl.ds(sid * ZERO_ROWS, ZERO_ROWS)],
                out_hbm.at[cid].at[pl.ds(sid * ZERO_ROWS, ZERO_ROWS)],
            )

        @pl.when(sid == NUM_SUBCORES - 1)
        def _():
            pltpu.sync_copy(
                acc.at[pl.ds(15 * ZERO_ROWS, LAST_ROWS)],
                out_hbm.at[cid].at[pl.ds(15 * ZERO_ROWS, LAST_ROWS)],
            )

    return seg_kernel(feat, src, dst, zeros)


def _tc_layer0(x, acc0, W_self0, W_neigh0, b0):
    R = 1000

    def body(x_ref, a_ref, ws_ref, wn_ref, b_ref, h_ref):
        agg = a_ref[0] + a_ref[1]
        h = (
            jnp.dot(x_ref[...], ws_ref[...], preferred_element_type=jnp.float32)
            + jnp.dot(agg, wn_ref[...], preferred_element_type=jnp.float32)
            + b_ref[...]
        )
        h_ref[...] = jnp.maximum(h, 0.0)

    return pl.pallas_call(
        body,
        grid=(N_NODES // R,),
        in_specs=[
            pl.BlockSpec((R, 128), lambda i: (i, 0)),
            pl.BlockSpec((2, R, 128), lambda i: (0, i, 0)),
            pl.BlockSpec((128, 128), lambda i: (0, 0)),
            pl.BlockSpec((128, 128), lambda i: (0, 0)),
            pl.BlockSpec((1, 128), lambda i: (0, 0)),
        ],
        out_specs=pl.BlockSpec((R, 128), lambda i: (i, 0)),
        out_shape=jax.ShapeDtypeStruct((N_NODES, 128), jnp.float32),
    )(x, acc0, W_self0, W_neigh0, b0.reshape(1, 128))


def _tc_layer1(h, acc1, W_self1, W_neigh1, b1):
    R = 1000

    def body(h_ref, a_ref, ws_ref, wn_ref, b_ref, out_ref):
        agg = a_ref[0] + a_ref[1]
        out_ref[...] = (
            jnp.dot(h_ref[...], ws_ref[...], preferred_element_type=jnp.float32)
            + jnp.dot(agg, wn_ref[...], preferred_element_type=jnp.float32)
            + b_ref[...]
        )

    return pl.pallas_call(
        body,
        grid=(N_NODES // R,),
        in_specs=[
            pl.BlockSpec((R, 128), lambda i: (i, 0)),
            pl.BlockSpec((2, R, 128), lambda i: (0, i, 0)),
            pl.BlockSpec((128, 64), lambda i: (0, 0)),
            pl.BlockSpec((128, 64), lambda i: (0, 0)),
            pl.BlockSpec((1, 64), lambda i: (0, 0)),
        ],
        out_specs=pl.BlockSpec((R, 64), lambda i: (i, 0)),
        out_shape=jax.ShapeDtypeStruct((N_NODES, 64), jnp.float32),
    )(h, acc1, W_self1, W_neigh1, b1.reshape(1, 64))


def _pad_edges(edge_index):
    e = edge_index.shape[1]
    pad = E_PAD - e
    pad_src = (jnp.arange(pad, dtype=jnp.int32) * 8) % N_NODES
    pad_dst = N_NODES + (jnp.arange(pad, dtype=jnp.int32) % (ACC_ROWS - N_NODES))
    src = jnp.concatenate([edge_index[0], pad_src])
    dst = jnp.concatenate([edge_index[1], pad_dst.astype(jnp.int32)])
    src = src.reshape(NUM_CORES, NUM_SUBCORES, E_PER_W)
    dst = dst.reshape(NUM_CORES, NUM_SUBCORES, CHUNKS_PER_W, CHUNK)
    return src, dst


def kernel(x, edge_index0, edge_index1, W_self0, W_neigh0, b0,
           W_self1, W_neigh1, b1):
    src0, dst0 = _pad_edges(edge_index0)
    src1, dst1 = _pad_edges(edge_index1)
    zeros128 = jnp.zeros((ZERO_ROWS, 128), jnp.float32)

    acc0 = _segsum_sc(x, src0, dst0, zeros128, 128)
    h = _tc_layer0(x, acc0, W_self0, W_neigh0, b0)
    acc1 = _segsum_sc(h, src1, dst1, zeros128, 128)
    return _tc_layer1(h, acc1, W_self1, W_neigh1, b1)
